# Initial kernel scaffold; baseline (speedup 1.0000x reference)
import jax
import jax.numpy as jnp
from jax.experimental import pallas as pl


def kernel(input, grid_x, grid_y):
    raise NotImplementedError("write your pallas kernel here")


# speedup vs baseline: 1.7043x; 1.7043x over previous
import functools

import jax
import jax.numpy as jnp
from jax import lax
from jax.experimental import pallas as pl
from jax.experimental.pallas import tpu as pltpu
from jax.experimental.pallas import tpu_sc as plsc

_BIG = 1 << 30


def _argmax_body(x_ref, idx_ref):
    x = x_ref[0]
    h, w = x.shape
    m = jnp.max(x)
    rowmax = jnp.max(x, axis=1, keepdims=True)
    riota = lax.broadcasted_iota(jnp.int32, (h, 1), 0)
    row = jnp.min(jnp.where(rowmax == m, riota, _BIG))
    rv = x_ref[0, pl.ds(row, 1), :]
    ciota = lax.broadcasted_iota(jnp.int32, (1, w), 1)
    col = jnp.min(jnp.where(rv == m, ciota, _BIG))
    idx_ref[0, 0, 0] = row * w + col


def _tc_argmax(x3):
    b, h, w = x3.shape
    return pl.pallas_call(
        _argmax_body,
        grid=(b,),
        in_specs=[pl.BlockSpec((1, h, w), lambda i: (i, 0, 0))],
        out_specs=pl.BlockSpec((1, 1, 1), lambda i: (i, 0, 0),
                               memory_space=pltpu.SMEM),
        out_shape=jax.ShapeDtypeStruct((b, 1, 1), jnp.int32),
        compiler_params=pltpu.CompilerParams(
            dimension_semantics=("arbitrary",)),
    )(x3)


def _make_sc_gather(b, n):
    chunk = 8
    nworkers = b // chunk
    mesh = plsc.VectorSubcoreMesh(core_axis_name="c", subcore_axis_name="s")

    @functools.partial(
        pl.kernel,
        mesh=mesh,
        out_type=[jax.ShapeDtypeStruct((b,), jnp.float32),
                  jax.ShapeDtypeStruct((b,), jnp.float32)],
        scratch_types=[pltpu.VMEM((chunk,), jnp.int32),
                       pltpu.VMEM((chunk,), jnp.float32),
                       pltpu.VMEM((chunk,), jnp.float32),
                       pltpu.SemaphoreType.DMA],
    )
    def gather_k(idx_hbm, gx_hbm, gy_hbm, ox_hbm, oy_hbm,
                 idx_v, x_v, y_v, sem):
        wid = lax.axis_index("s") * 2 + lax.axis_index("c")

        @pl.when(wid < nworkers)
        def _():
            base = wid * chunk
            pltpu.sync_copy(idx_hbm.at[pl.ds(base, chunk)], idx_v)
            pltpu.async_copy(gx_hbm.at[idx_v], x_v, sem).wait()
            pltpu.async_copy(gy_hbm.at[idx_v], y_v, sem).wait()
            pltpu.sync_copy(x_v, ox_hbm.at[pl.ds(base, chunk)])
            pltpu.sync_copy(y_v, oy_hbm.at[pl.ds(base, chunk)])

    return gather_k


def kernel(input, grid_x, grid_y):
    b = input.shape[0]
    h, w = input.shape[2], input.shape[3]
    n = h * w
    x3 = input.reshape(b, h, w)
    idx = _tc_argmax(x3).reshape(b)
    gx = grid_x.reshape(n)
    gy = grid_y.reshape(n)
    ox, oy = _make_sc_gather(b, n)(idx, gx, gy)
    return jnp.concatenate((ox[:, None], oy[:, None]), axis=1)


# speedup vs baseline: 2.9667x; 1.7407x over previous
import functools

import jax
import jax.numpy as jnp
from jax import lax
from jax.experimental import pallas as pl
from jax.experimental.pallas import tpu as pltpu
from jax.experimental.pallas import tpu_sc as plsc

_BIG = 1 << 30


_BB = 4


def _argmax_body(x_ref, idx_ref):
    bb, h, w = x_ref.shape
    ciota = lax.broadcasted_iota(jnp.int32, (1, w), 1)
    for s in range(bb):
        x = x_ref[s]
        m = jnp.max(x)
        rowmax = jnp.max(x, axis=1, keepdims=True)
        riota = lax.broadcasted_iota(jnp.int32, (h, 1), 0)
        row = jnp.min(jnp.where(rowmax == m, riota, _BIG))
        rv = x_ref[s, pl.ds(row, 1), :]
        col = jnp.min(jnp.where(rv == m, ciota, _BIG))
        idx_ref[s, 0, 0] = row * w + col


def _tc_argmax(x3):
    b, h, w = x3.shape
    return pl.pallas_call(
        _argmax_body,
        grid=(b // _BB,),
        in_specs=[pl.BlockSpec((_BB, h, w), lambda i: (i, 0, 0))],
        out_specs=pl.BlockSpec((_BB, 1, 1), lambda i: (i, 0, 0),
                               memory_space=pltpu.SMEM),
        out_shape=jax.ShapeDtypeStruct((b, 1, 1), jnp.int32),
        compiler_params=pltpu.CompilerParams(
            dimension_semantics=("arbitrary",)),
    )(x3)


def _make_sc_gather(b, n):
    chunk = 8
    nworkers = b // chunk
    mesh = plsc.VectorSubcoreMesh(core_axis_name="c", subcore_axis_name="s")

    @functools.partial(
        pl.kernel,
        mesh=mesh,
        out_type=[jax.ShapeDtypeStruct((b,), jnp.float32),
                  jax.ShapeDtypeStruct((b,), jnp.float32)],
        scratch_types=[pltpu.VMEM((chunk,), jnp.int32),
                       pltpu.VMEM((chunk,), jnp.float32),
                       pltpu.VMEM((chunk,), jnp.float32),
                       pltpu.SemaphoreType.DMA],
    )
    def gather_k(idx_hbm, gx_hbm, gy_hbm, ox_hbm, oy_hbm,
                 idx_v, x_v, y_v, sem):
        wid = lax.axis_index("s") * 2 + lax.axis_index("c")

        @pl.when(wid < nworkers)
        def _():
            base = wid * chunk
            pltpu.sync_copy(idx_hbm.at[pl.ds(base, chunk)], idx_v)
            pltpu.async_copy(gx_hbm.at[idx_v], x_v, sem).wait()
            pltpu.async_copy(gy_hbm.at[idx_v], y_v, sem).wait()
            pltpu.sync_copy(x_v, ox_hbm.at[pl.ds(base, chunk)])
            pltpu.sync_copy(y_v, oy_hbm.at[pl.ds(base, chunk)])

    return gather_k


def kernel(input, grid_x, grid_y):
    b = input.shape[0]
    h, w = input.shape[2], input.shape[3]
    n = h * w
    x3 = input.reshape(b, h, w)
    idx = _tc_argmax(x3).reshape(b)
    gx = grid_x.reshape(n)
    gy = grid_y.reshape(n)
    ox, oy = _make_sc_gather(b, n)(idx, gx, gy)
    return jnp.concatenate((ox[:, None], oy[:, None]), axis=1)


# speedup vs baseline: 3.3213x; 1.1195x over previous
import functools

import jax
import jax.numpy as jnp
from jax import lax
from jax.experimental import pallas as pl
from jax.experimental.pallas import tpu as pltpu
from jax.experimental.pallas import tpu_sc as plsc

_BIG = 1 << 30


_BB = 8


def _argmax_body(x_ref, idx_ref):
    bb, h, w = x_ref.shape
    ciota = lax.broadcasted_iota(jnp.int32, (1, w), 1)
    for s in range(bb):
        x = x_ref[s]
        m = jnp.max(x)
        rowmax = jnp.max(x, axis=1, keepdims=True)
        riota = lax.broadcasted_iota(jnp.int32, (h, 1), 0)
        row = jnp.min(jnp.where(rowmax == m, riota, _BIG))
        rv = x_ref[s, pl.ds(row, 1), :]
        col = jnp.min(jnp.where(rv == m, ciota, _BIG))
        idx_ref[s, 0, 0] = row * w + col


def _tc_argmax(x3):
    b, h, w = x3.shape
    return pl.pallas_call(
        _argmax_body,
        grid=(b // _BB,),
        in_specs=[pl.BlockSpec((_BB, h, w), lambda i: (i, 0, 0))],
        out_specs=pl.BlockSpec((_BB, 1, 1), lambda i: (i, 0, 0),
                               memory_space=pltpu.SMEM),
        out_shape=jax.ShapeDtypeStruct((b, 1, 1), jnp.int32),
        compiler_params=pltpu.CompilerParams(
            dimension_semantics=("arbitrary",)),
    )(x3)


def _make_sc_gather(b, n):
    chunk = 8
    nworkers = b // chunk
    mesh = plsc.VectorSubcoreMesh(core_axis_name="c", subcore_axis_name="s")

    @functools.partial(
        pl.kernel,
        mesh=mesh,
        out_type=[jax.ShapeDtypeStruct((b,), jnp.float32),
                  jax.ShapeDtypeStruct((b,), jnp.float32)],
        scratch_types=[pltpu.VMEM((chunk,), jnp.int32),
                       pltpu.VMEM((chunk,), jnp.float32),
                       pltpu.VMEM((chunk,), jnp.float32),
                       pltpu.SemaphoreType.DMA],
    )
    def gather_k(idx_hbm, gx_hbm, gy_hbm, ox_hbm, oy_hbm,
                 idx_v, x_v, y_v, sem):
        wid = lax.axis_index("s") * 2 + lax.axis_index("c")

        @pl.when(wid < nworkers)
        def _():
            base = wid * chunk
            pltpu.sync_copy(idx_hbm.at[pl.ds(base, chunk)], idx_v)
            pltpu.async_copy(gx_hbm.at[idx_v], x_v, sem).wait()
            pltpu.async_copy(gy_hbm.at[idx_v], y_v, sem).wait()
            pltpu.sync_copy(x_v, ox_hbm.at[pl.ds(base, chunk)])
            pltpu.sync_copy(y_v, oy_hbm.at[pl.ds(base, chunk)])

    return gather_k


def kernel(input, grid_x, grid_y):
    b = input.shape[0]
    h, w = input.shape[2], input.shape[3]
    n = h * w
    x3 = input.reshape(b, h, w)
    idx = _tc_argmax(x3).reshape(b)
    gx = grid_x.reshape(n)
    gy = grid_y.reshape(n)
    ox, oy = _make_sc_gather(b, n)(idx, gx, gy)
    return jnp.concatenate((ox[:, None], oy[:, None]), axis=1)


# speedup vs baseline: 3.5159x; 1.0586x over previous
import functools

import jax
import jax.numpy as jnp
from jax import lax
from jax.experimental import pallas as pl
from jax.experimental.pallas import tpu as pltpu
from jax.experimental.pallas import tpu_sc as plsc

_BIG = 1 << 30


_BB = 16


def _argmax_body(x_ref, idx_ref):
    bb, h, w = x_ref.shape
    ciota = lax.broadcasted_iota(jnp.int32, (1, w), 1)
    for s in range(bb):
        x = x_ref[s]
        m = jnp.max(x)
        rowmax = jnp.max(x, axis=1, keepdims=True)
        riota = lax.broadcasted_iota(jnp.int32, (h, 1), 0)
        row = jnp.min(jnp.where(rowmax == m, riota, _BIG))
        rv = x_ref[s, pl.ds(row, 1), :]
        col = jnp.min(jnp.where(rv == m, ciota, _BIG))
        idx_ref[s, 0, 0] = row * w + col


def _tc_argmax(x3):
    b, h, w = x3.shape
    return pl.pallas_call(
        _argmax_body,
        grid=(b // _BB,),
        in_specs=[pl.BlockSpec((_BB, h, w), lambda i: (i, 0, 0))],
        out_specs=pl.BlockSpec((_BB, 1, 1), lambda i: (i, 0, 0),
                               memory_space=pltpu.SMEM),
        out_shape=jax.ShapeDtypeStruct((b, 1, 1), jnp.int32),
        compiler_params=pltpu.CompilerParams(
            dimension_semantics=("arbitrary",)),
    )(x3)


def _make_sc_gather(b, n):
    chunk = 8
    nworkers = b // chunk
    mesh = plsc.VectorSubcoreMesh(core_axis_name="c", subcore_axis_name="s")

    @functools.partial(
        pl.kernel,
        mesh=mesh,
        out_type=[jax.ShapeDtypeStruct((b,), jnp.float32),
                  jax.ShapeDtypeStruct((b,), jnp.float32)],
        scratch_types=[pltpu.VMEM((chunk,), jnp.int32),
                       pltpu.VMEM((chunk,), jnp.float32),
                       pltpu.VMEM((chunk,), jnp.float32),
                       pltpu.SemaphoreType.DMA],
    )
    def gather_k(idx_hbm, gx_hbm, gy_hbm, ox_hbm, oy_hbm,
                 idx_v, x_v, y_v, sem):
        wid = lax.axis_index("s") * 2 + lax.axis_index("c")

        @pl.when(wid < nworkers)
        def _():
            base = wid * chunk
            pltpu.sync_copy(idx_hbm.at[pl.ds(base, chunk)], idx_v)
            pltpu.async_copy(gx_hbm.at[idx_v], x_v, sem).wait()
            pltpu.async_copy(gy_hbm.at[idx_v], y_v, sem).wait()
            pltpu.sync_copy(x_v, ox_hbm.at[pl.ds(base, chunk)])
            pltpu.sync_copy(y_v, oy_hbm.at[pl.ds(base, chunk)])

    return gather_k


def kernel(input, grid_x, grid_y):
    b = input.shape[0]
    h, w = input.shape[2], input.shape[3]
    n = h * w
    x3 = input.reshape(b, h, w)
    idx = _tc_argmax(x3).reshape(b)
    gx = grid_x.reshape(n)
    gy = grid_y.reshape(n)
    ox, oy = _make_sc_gather(b, n)(idx, gx, gy)
    return jnp.concatenate((ox[:, None], oy[:, None]), axis=1)


# speedup vs baseline: 3.5592x; 1.0123x over previous
import functools

import jax
import jax.numpy as jnp
from jax import lax
from jax.experimental import pallas as pl
from jax.experimental.pallas import tpu as pltpu
from jax.experimental.pallas import tpu_sc as plsc

_BIG = 1 << 30


_BB = 16


def _argmax_body(x_ref, idx_ref):
    bb, h, w = x_ref.shape
    ciota = lax.broadcasted_iota(jnp.int32, (1, w), 1)
    for s in range(bb):
        rowmax = jnp.max(x_ref[s], axis=1, keepdims=True)
        m = jnp.max(rowmax)
        riota = lax.broadcasted_iota(jnp.int32, (h, 1), 0)
        row = jnp.min(jnp.where(rowmax == m, riota, _BIG))
        rv = x_ref[s, pl.ds(row, 1), :]
        col = jnp.min(jnp.where(rv == m, ciota, _BIG))
        idx_ref[s, 0, 0] = row * w + col


def _tc_argmax(x3):
    b, h, w = x3.shape
    return pl.pallas_call(
        _argmax_body,
        grid=(b // _BB,),
        in_specs=[pl.BlockSpec((_BB, h, w), lambda i: (i, 0, 0))],
        out_specs=pl.BlockSpec((_BB, 1, 1), lambda i: (i, 0, 0),
                               memory_space=pltpu.SMEM),
        out_shape=jax.ShapeDtypeStruct((b, 1, 1), jnp.int32),
        compiler_params=pltpu.CompilerParams(
            dimension_semantics=("arbitrary",)),
    )(x3)


def _make_sc_gather(b, n):
    chunk = 8
    nworkers = b // chunk
    mesh = plsc.VectorSubcoreMesh(core_axis_name="c", subcore_axis_name="s")

    @functools.partial(
        pl.kernel,
        mesh=mesh,
        out_type=[jax.ShapeDtypeStruct((b,), jnp.float32),
                  jax.ShapeDtypeStruct((b,), jnp.float32)],
        scratch_types=[pltpu.VMEM((chunk,), jnp.int32),
                       pltpu.VMEM((chunk,), jnp.float32),
                       pltpu.VMEM((chunk,), jnp.float32),
                       pltpu.SemaphoreType.DMA],
    )
    def gather_k(idx_hbm, gx_hbm, gy_hbm, ox_hbm, oy_hbm,
                 idx_v, x_v, y_v, sem):
        wid = lax.axis_index("s") * 2 + lax.axis_index("c")

        @pl.when(wid < nworkers)
        def _():
            base = wid * chunk
            pltpu.sync_copy(idx_hbm.at[pl.ds(base, chunk)], idx_v)
            pltpu.async_copy(gx_hbm.at[idx_v], x_v, sem).wait()
            pltpu.async_copy(gy_hbm.at[idx_v], y_v, sem).wait()
            pltpu.sync_copy(x_v, ox_hbm.at[pl.ds(base, chunk)])
            pltpu.sync_copy(y_v, oy_hbm.at[pl.ds(base, chunk)])

    return gather_k


def kernel(input, grid_x, grid_y):
    b = input.shape[0]
    h, w = input.shape[2], input.shape[3]
    n = h * w
    x3 = input.reshape(b, h, w)
    idx = _tc_argmax(x3).reshape(b)
    gx = grid_x.reshape(n)
    gy = grid_y.reshape(n)
    ox, oy = _make_sc_gather(b, n)(idx, gx, gy)
    return jnp.concatenate((ox[:, None], oy[:, None]), axis=1)
